# baseline (speedup 1.0000x reference)
import jax
import jax.numpy as jnp
from jax import lax
from jax.experimental import pallas as pl
from jax.experimental.pallas import tpu as pltpu
from jax.experimental.pallas import tpu_sc as plsc

N_SEG = 100000
NP = 100096
BE = 1024
NW = 32
NT = 16
NPW = NP // NW
SPAN1 = 256
SPAN2 = 2048
STAB = NP + SPAN2 + 16

_mesh = plsc.VectorSubcoreMesh(core_axis_name="c", subcore_axis_name="s")
_SC_PARAMS = pltpu.CompilerParams(
    use_tc_tiling_on_sc=False, needs_layout_passes=False)


def _scalar_at(vec16, pos):
    lane = lax.iota(jnp.int32, 16)
    return jnp.sum(jnp.where(lane == pos, vec16, 0))


def _phase_a(scores3, dst2, bounds, zerosw):
    def body(scores_hbm, dst_hbm, bounds_hbm, zeros_hbm,
             osum, osq, ocnt,
             tsum, tsq, tcnt, sco_v, dst_v, bnd_v):
        cid = lax.axis_index("c")
        sid = lax.axis_index("s")
        w = sid * 2 + cid
        pltpu.sync_copy(zeros_hbm, tsum)
        pltpu.sync_copy(zeros_hbm, tsq)
        pltpu.sync_copy(zeros_hbm, tcnt)
        pltpu.sync_copy(bounds_hbm.at[w], bnd_v)
        bv = bnd_v[...]
        blk0 = _scalar_at(bv, 0)
        nblk = _scalar_at(bv, 1)
        base = w * NPW

        lane = lax.iota(jnp.int32, 16)
        cvec = lax.rem(lane, 4)
        rv0 = lax.div(lane, 4)
        onesf = jnp.full((16,), 1.0, jnp.float32)

        def block_body(i, _):
            b = blk0 + i
            pltpu.sync_copy(scores_hbm.at[b], sco_v)
            pltpu.sync_copy(dst_hbm.at[b], dst_v)

            def group_body(g, _):
                for k in range(4):
                    vals = sco_v[pl.ds(g * 64 + k * 16, 16)]
                    relg = plsc.load_gather(dst_v, [g * 16 + rv0 + 4 * k]) - base
                    mk = jnp.logical_and(relg >= 0, relg < NPW)
                    plsc.addupdate_scatter(tsum, [relg, cvec], vals, mask=mk)
                    plsc.addupdate_scatter(tsq, [relg, cvec], vals * vals,
                                           mask=mk)
                    plsc.addupdate_scatter(tcnt, [relg, cvec], onesf, mask=mk)
                return 0

            lax.fori_loop(0, BE // 16, group_body, 0, unroll=2)
            return 0

        lax.fori_loop(0, nblk, block_body, 0)
        rows = pl.ds(base, NPW)
        pltpu.sync_copy(tsum, osum.at[rows, :])
        pltpu.sync_copy(tsq, osq.at[rows, :])
        pltpu.sync_copy(tcnt, ocnt.at[rows, :])

    f = pl.kernel(
        body,
        out_type=[jax.ShapeDtypeStruct((NP, 4), jnp.float32)] * 3,
        mesh=_mesh,
        compiler_params=_SC_PARAMS,
        scratch_types=[
            pltpu.VMEM((NPW, 4), jnp.float32),
            pltpu.VMEM((NPW, 4), jnp.float32),
            pltpu.VMEM((NPW, 4), jnp.float32),
            pltpu.VMEM((BE * 4,), jnp.float32),
            pltpu.VMEM((BE,), jnp.int32),
            pltpu.VMEM((16,), jnp.int32),
        ],
    )
    return f(scores3, dst2, bounds, zerosw)


def _phase_b_body(ps, pq, pc, g, b, oa, ob):
    s = ps[...]
    q = pq[...]
    c = pc[...]
    mean = s / jnp.maximum(c, 1.0)
    var = jnp.maximum(q - c * mean * mean, 0.0)
    std = jnp.sqrt(var / jnp.maximum(c, 1.0))
    inv = 1.0 / jnp.maximum(std, 1e-5)
    a = g[0:1, :] * inv
    oa[...] = a
    ob[...] = b[0:1, :] - mean * a


def _phase_b(psum, psq, pcnt, gvec, bvec):
    rows = NP * 4 // 128
    f = pl.pallas_call(
        _phase_b_body,
        out_shape=[jax.ShapeDtypeStruct((rows, 128), jnp.float32)] * 2,
    )
    return f(psum.reshape(rows, 128), psq.reshape(rows, 128),
             pcnt.reshape(rows, 128), gvec, bvec)


def _phase_c(scores3, dst2, atab, btab):
    nblocks = scores3.shape[0]

    def body(scores_hbm, dst_hbm, a_hbm, b_hbm, out_hbm,
             sco_v, ga_v, gb_v, out_v, dst_v, idx_v):
        cid = lax.axis_index("c")
        sid = lax.axis_index("s")
        w = sid * 2 + cid

        lane = lax.iota(jnp.int32, 16)
        cvec = lax.rem(lane, 4)
        rv0 = lax.div(lane, 4)
        dummy = jnp.full((16,), N_SEG, jnp.int32)
        for j in range(8):
            for t in range(8):
                idx_v[j, pl.ds(128 + 16 * t, 16)] = dummy

        nfull = nblocks // NW
        extra = (w < (nblocks - nfull * NW)).astype(jnp.int32)

        def block_body(i, _):
            b = w + NW * i
            pltpu.sync_copy(scores_hbm.at[b], sco_v)
            pltpu.sync_copy(dst_hbm.at[b], dst_v)
            first = lax.reduce_min(dst_v[pl.ds(0, 16)], (0,))
            last = lax.reduce_max(dst_v[pl.ds(BE - 16, 16)], (0,))
            first8 = lax.div(first, 8) * 8
            span = last - first8 + 1

            def run_linear(nrows):
                pltpu.sync_copy(a_hbm.at[pl.ds(first8, nrows), :],
                                ga_v.at[pl.ds(0, nrows), :])
                pltpu.sync_copy(b_hbm.at[pl.ds(first8, nrows), :],
                                gb_v.at[pl.ds(0, nrows), :])

                def fma_body(g, _):
                    for k in range(4):
                        vals = sco_v[pl.ds(g * 64 + k * 16, 16)]
                        relg = (plsc.load_gather(dst_v, [g * 16 + rv0 + 4 * k])
                                - first8)
                        av = plsc.load_gather(ga_v, [relg, cvec])
                        bv2 = plsc.load_gather(gb_v, [relg, cvec])
                        out_v[pl.ds(g * 64 + k * 16, 16)] = av * vals + bv2
                    return 0

                lax.fori_loop(0, BE // 16, fma_body, 0, unroll=2)

            def run_fallback():
                for j in range(8):
                    for t in range(8):
                        idx_v[j, pl.ds(16 * t, 16)] = dst_v[
                            pl.ds(j * 128 + 16 * t, 16)]
                for j in range(8):
                    pltpu.sync_copy(a_hbm.at[idx_v.at[j]],
                                    ga_v.at[pl.ds(j * 256, 256), :])
                    pltpu.sync_copy(b_hbm.at[idx_v.at[j]],
                                    gb_v.at[pl.ds(j * 256, 256), :])

                def fma_body(g, _):
                    for k in range(4):
                        vals = sco_v[pl.ds(g * 64 + k * 16, 16)]
                        ev = g * 16 + rv0 + 4 * k
                        rowv = ev + lax.div(ev, 128) * 128
                        av = plsc.load_gather(ga_v, [rowv, cvec])
                        bv2 = plsc.load_gather(gb_v, [rowv, cvec])
                        out_v[pl.ds(g * 64 + k * 16, 16)] = av * vals + bv2
                    return 0

                lax.fori_loop(0, BE // 16, fma_body, 0, unroll=2)

            @pl.when(span <= SPAN1)
            def _():
                run_linear(SPAN1 + 8)

            @pl.when(jnp.logical_and(span > SPAN1, span <= SPAN2))
            def _():
                run_linear(SPAN2 + 8)

            @pl.when(span > SPAN2)
            def _():
                run_fallback()

            pltpu.sync_copy(out_v, out_hbm.at[b])
            return 0

        lax.fori_loop(0, nfull + extra, block_body, 0)

    f = pl.kernel(
        body,
        out_type=jax.ShapeDtypeStruct((nblocks, BE * 4), jnp.float32),
        mesh=_mesh,
        compiler_params=_SC_PARAMS,
        scratch_types=[
            pltpu.VMEM((BE * 4,), jnp.float32),
            pltpu.VMEM((SPAN2 + 16, 4), jnp.float32),
            pltpu.VMEM((SPAN2 + 16, 4), jnp.float32),
            pltpu.VMEM((BE * 4,), jnp.float32),
            pltpu.VMEM((BE,), jnp.int32),
            pltpu.VMEM((8, 256), jnp.int32),
        ],
    )
    return f(scores3, dst2, atab, btab)


def kernel(edge_scores, dst, gain, bias):
    e, h, _ = edge_scores.shape
    nblocks = e // BE
    scores3 = edge_scores.reshape(nblocks, BE * h)
    dst2 = dst.reshape(nblocks, BE)
    node_bnd = jnp.arange(NW + 1, dtype=jnp.int32) * NPW
    edge_bnd = jnp.searchsorted(dst, node_bnd, side="left").astype(jnp.int32)
    blk0 = edge_bnd[:-1] // BE
    blkend = (edge_bnd[1:] + BE - 1) // BE
    nblk = jnp.maximum(blkend - blk0, 0)
    blk0 = jnp.minimum(blk0, nblocks - 1)
    bounds = jnp.zeros((NW, 16), jnp.int32)
    bounds = bounds.at[:, 0].set(blk0).at[:, 1].set(nblk)
    zerosw = jnp.zeros((NPW, 4), jnp.float32)

    psum, psq, pcnt = _phase_a(scores3, dst2, bounds, zerosw)
    gvec = jnp.tile(gain.reshape(1, h), (1, 128 // h))
    bvec = jnp.tile(bias.reshape(1, h), (1, 128 // h))
    a2, b2 = _phase_b(psum, psq, pcnt, gvec, bvec)
    pad = jnp.zeros((STAB - NP, 4), jnp.float32)
    atab = jnp.concatenate([a2.reshape(NP, 4), pad])
    btab = jnp.concatenate([b2.reshape(NP, 4), pad])
    out3 = _phase_c(scores3, dst2, atab, btab)
    return out3.reshape(e, h, 1)
